# Initial kernel scaffold; baseline (speedup 1.0000x reference)
import jax
import jax.numpy as jnp
from jax import lax
from jax.experimental import pallas as pl
from jax.experimental.pallas import tpu as pltpu
from jax.experimental.pallas import tpu_sc as plsc

N = 10000
E = 160000
D = 256
HD = 256
H = 8
OUT = 256

NC = 2
NS = 16
NW = NC * NS
L = 16

NPW = (N + NW - 1) // NW
MP = ((NPW + 7) // 8) * 8
C = 8
LPAD = E + C * N
ROW = H * HD
KVROW = 2 * ROW


def _mm_body(x_ref, w_ref, b_ref, o_ref):
    o_ref[...] = (
        jnp.dot(x_ref[...], w_ref[...], preferred_element_type=jnp.float32)
        + b_ref[...]
    )


def _matmul(x, w, b, bm=1024, bn=1024):
    m, k = x.shape
    _, n = w.shape
    bn = min(bn, n)
    grid = (pl.cdiv(m, bm), pl.cdiv(n, bn))
    return pl.pallas_call(
        _mm_body,
        grid=grid,
        in_specs=[
            pl.BlockSpec((bm, k), lambda i, j: (i, 0)),
            pl.BlockSpec((k, bn), lambda i, j: (0, j)),
            pl.BlockSpec((1, bn), lambda i, j: (0, j)),
        ],
        out_specs=pl.BlockSpec((bm, bn), lambda i, j: (i, j)),
        out_shape=jax.ShapeDtypeStruct((m, n), jnp.float32),
    )(x, w, b.reshape(1, n))


def _edge_body(q_hbm, kv_hbm, skip_hbm, srcpad_hbm, meta_hbm, out_hbm,
               qrow_v, kvbuf_v, idx_v, outrow_v, acc_v, den_v, meta_v, sem):
    cid = lax.axis_index("c")
    sid = lax.axis_index("s")
    wid = sid * NC + cid
    n0 = wid * NPW
    nlocal = jnp.maximum(jnp.minimum(NPW, N - n0), 0)

    pltpu.sync_copy(meta_hbm.at[wid], meta_v)

    def node_loop(i, _):
        n = n0 + i
        base = meta_v[i]
        deg = meta_v[MP + i]

        pltpu.sync_copy(q_hbm.at[n], qrow_v)
        pltpu.sync_copy(skip_hbm.at[n], outrow_v)

        zero = jnp.zeros((L,), jnp.float32)
        for t in range(ROW // L):
            acc_v[pl.ds(L * t, L)] = zero
        for h in range(H):
            den_v[pl.ds(L * h, L)] = zero

        nchunks = (deg + (C - 1)) // C

        def chunk_loop(c, _):
            pos = base + c * C
            pltpu.sync_copy(srcpad_hbm.at[pl.ds(pos, C)], idx_v)
            pltpu.sync_copy(kv_hbm.at[idx_v], kvbuf_v)
            cnt = jnp.minimum(deg - c * C, C)

            for h in range(H):
                qh = [qrow_v[pl.ds(h * HD + L * t, L)] for t in range(HD // L)]
                acch = [acc_v[pl.ds(h * HD + L * t, L)] for t in range(HD // L)]
                denh = den_v[pl.ds(L * h, L)]

                def edge_body(j, carry, h=h, qh=qh):
                    acch, denh = carry
                    dot = qh[0] * kvbuf_v[j, pl.ds(h * HD, L)]
                    for t in range(1, HD // L):
                        dot = dot + qh[t] * kvbuf_v[j, pl.ds(h * HD + L * t, L)]
                    s = jnp.sum(dot)
                    p = jnp.exp(jnp.full((L,), s, jnp.float32))
                    denh = denh + p
                    acch = [
                        acch[t] + p * kvbuf_v[j, pl.ds(ROW + h * HD + L * t, L)]
                        for t in range(HD // L)
                    ]
                    return (acch, denh)

                acch, denh = lax.fori_loop(0, cnt, edge_body, (acch, denh))
                for t in range(HD // L):
                    acc_v[pl.ds(h * HD + L * t, L)] = acch[t]
                den_v[pl.ds(L * h, L)] = denh
            return 0

        lax.fori_loop(0, nchunks, chunk_loop, 0)

        for t in range(HD // L):
            o = outrow_v[pl.ds(L * t, L)]
            tot = jnp.zeros((L,), jnp.float32)
            for h in range(H):
                a = acc_v[pl.ds(h * HD + L * t, L)]
                d = den_v[pl.ds(L * h, L)]
                tot = tot + a / (d + 1e-16)
            o = o + tot * (1.0 / H)
            outrow_v[pl.ds(L * t, L)] = jnp.maximum(o, 0.0)
        pltpu.sync_copy(outrow_v, out_hbm.at[n])
        return 0

    lax.fori_loop(0, nlocal, node_loop, 0)


def _edge_stage(q, kv, skip, srcpad, meta):
    mesh = plsc.VectorSubcoreMesh(core_axis_name="c", subcore_axis_name="s")
    f = pl.kernel(
        _edge_body,
        out_type=jax.ShapeDtypeStruct((N, HD), jnp.float32),
        mesh=mesh,
        scratch_types=[
            pltpu.VMEM((ROW,), jnp.float32),
            pltpu.VMEM((C, KVROW), jnp.float32),
            pltpu.VMEM((C,), jnp.int32),
            pltpu.VMEM((HD,), jnp.float32),
            pltpu.VMEM((ROW,), jnp.float32),
            pltpu.VMEM((H * L,), jnp.float32),
            pltpu.VMEM((2 * MP,), jnp.int32),
            pltpu.SemaphoreType.DMA,
        ],
    )
    return f(q, kv, skip, srcpad, meta)


def _prep_edges(edge_index):
    src = edge_index[0]
    dst = edge_index[1]
    order = jnp.argsort(dst)
    src_s = src[order]
    dst_s = dst[order]
    offs = jnp.searchsorted(dst_s, jnp.arange(N + 1, dtype=jnp.int32)).astype(jnp.int32)
    deg = offs[1:] - offs[:-1]
    cap = ((deg + (C - 1)) // C) * C
    offs_pad = jnp.concatenate(
        [jnp.zeros((1,), jnp.int32), jnp.cumsum(cap).astype(jnp.int32)]
    )
    pos = offs_pad[dst_s] + (jnp.arange(E, dtype=jnp.int32) - offs[dst_s])
    srcpad = jnp.zeros((LPAD,), jnp.int32).at[pos].set(src_s)

    def per_worker(a):
        a = jnp.pad(a, (0, NW * NPW - N)).reshape(NW, NPW)
        return jnp.pad(a, ((0, 0), (0, MP - NPW)))

    meta = jnp.concatenate(
        [per_worker(offs_pad[:N]), per_worker(deg)], axis=1
    )
    return srcpad, meta


def _layer(x, srcpad, meta, Wq, bq, Wk, bk, Wv, bv, Ws, bs):
    scale = 1.0 / (HD ** 0.5)
    q = _matmul(x, Wq * scale, bq * scale)
    kv = _matmul(x, jnp.concatenate([Wk, Wv], axis=1),
                 jnp.concatenate([bk, bv], axis=0))
    skip = _matmul(x, Ws, bs, bn=256)
    return _edge_stage(q, kv, skip, srcpad, meta)


def kernel(x, edge_index, Wq0, bq0, Wk0, bk0, Wv0, bv0, Ws0, bs0,
           Wq1, bq1, Wk1, bk1, Wv1, bv1, Ws1, bs1, Wout, bout):
    srcpad, meta = _prep_edges(edge_index)
    h = _layer(x, srcpad, meta, Wq0, bq0, Wk0, bk0, Wv0, bv0, Ws0, bs0)
    h = _layer(h, srcpad, meta, Wq1, bq1, Wk1, bk1, Wv1, bv1, Ws1, bs1)
    return _matmul(h, Wout, bout, bn=256)


# speedup vs baseline: 2.6227x; 2.6227x over previous
import jax
import jax.numpy as jnp
from jax import lax
from jax.experimental import pallas as pl
from jax.experimental.pallas import tpu as pltpu
from jax.experimental.pallas import tpu_sc as plsc

N = 10000
E = 160000
D = 256
HD = 256
H = 8
OUT = 256

NC = 2
NS = 16
NW = NC * NS
L = 16

NPW = (N + NW - 1) // NW
MP = ((NPW + 7) // 8) * 8
C = 8
LPAD = E + C * N
ROW = H * HD
KVROW = 2 * ROW


def _mm_body(x_ref, w_ref, b_ref, o_ref):
    o_ref[...] = (
        jnp.dot(x_ref[...], w_ref[...], preferred_element_type=jnp.float32)
        + b_ref[...]
    )


def _matmul(x, w, b, bm=1024, bn=1024):
    m, k = x.shape
    _, n = w.shape
    bn = min(bn, n)
    grid = (pl.cdiv(m, bm), pl.cdiv(n, bn))
    return pl.pallas_call(
        _mm_body,
        grid=grid,
        in_specs=[
            pl.BlockSpec((bm, k), lambda i, j: (i, 0)),
            pl.BlockSpec((k, bn), lambda i, j: (0, j)),
            pl.BlockSpec((1, bn), lambda i, j: (0, j)),
        ],
        out_specs=pl.BlockSpec((bm, bn), lambda i, j: (i, j)),
        out_shape=jax.ShapeDtypeStruct((m, n), jnp.float32),
    )(x, w, b.reshape(1, n))


def _lane_sum(v):
    i = lax.iota(jnp.int32, L)
    for s in (8, 4, 2, 1):
        idx = lax.bitwise_and(i + s, L - 1)
        v = v + v.at[idx].get(mode="promise_in_bounds")
    return v

def _edge_body(q_hbm, kv_hbm, skip_hbm, srcpad_hbm, meta_hbm, out_hbm,
               qrow_v, kvbuf_v, idx_v, outrow_v, acc_v, den_v, meta_v, sem):
    cid = lax.axis_index("c")
    sid = lax.axis_index("s")
    wid = sid * NC + cid
    n0 = wid * NPW
    nlocal = jnp.maximum(jnp.minimum(NPW, N - n0), 0)

    pltpu.sync_copy(meta_hbm.at[wid], meta_v)

    def node_loop(i, _):
        n = n0 + i
        base = meta_v[pl.ds(i, L)][0]
        deg = meta_v[pl.ds(MP + i, L)][0]

        pltpu.sync_copy(q_hbm.at[n], qrow_v)
        pltpu.sync_copy(skip_hbm.at[n], outrow_v)

        zero = jnp.zeros((L,), jnp.float32)
        for t in range(ROW // L):
            acc_v[pl.ds(L * t, L)] = zero
        for h in range(H):
            den_v[pl.ds(L * h, L)] = zero

        nchunks = (deg + (C - 1)) // C

        def chunk_loop(c, _):
            pos = pl.multiple_of(base + c * C, 8)
            pltpu.sync_copy(srcpad_hbm.at[pl.ds(pos, C)], idx_v)
            pltpu.sync_copy(kv_hbm.at[idx_v], kvbuf_v)
            cnt = jnp.minimum(deg - c * C, C)

            for h in range(H):
                qh = [qrow_v[pl.ds(h * HD + L * t, L)] for t in range(HD // L)]
                acch = [acc_v[pl.ds(h * HD + L * t, L)] for t in range(HD // L)]
                denh = den_v[pl.ds(L * h, L)]

                def edge_body(j, carry, h=h, qh=qh):
                    acch, denh = carry
                    dot = qh[0] * kvbuf_v[j, pl.ds(h * HD, L)]
                    for t in range(1, HD // L):
                        dot = dot + qh[t] * kvbuf_v[j, pl.ds(h * HD + L * t, L)]
                    p = jnp.exp(_lane_sum(dot))
                    denh = denh + p
                    acch = [
                        acch[t] + p * kvbuf_v[j, pl.ds(ROW + h * HD + L * t, L)]
                        for t in range(HD // L)
                    ]
                    return (acch, denh)

                acch, denh = lax.fori_loop(0, cnt, edge_body, (acch, denh))
                for t in range(HD // L):
                    acc_v[pl.ds(h * HD + L * t, L)] = acch[t]
                den_v[pl.ds(L * h, L)] = denh
            return 0

        lax.fori_loop(0, nchunks, chunk_loop, 0)

        for t in range(HD // L):
            o = outrow_v[pl.ds(L * t, L)]
            tot = jnp.zeros((L,), jnp.float32)
            for h in range(H):
                a = acc_v[pl.ds(h * HD + L * t, L)]
                d = den_v[pl.ds(L * h, L)]
                tot = tot + a / (d + 1e-16)
            o = o + tot * (1.0 / H)
            outrow_v[pl.ds(L * t, L)] = jnp.maximum(o, 0.0)
        pltpu.sync_copy(outrow_v, out_hbm.at[n])
        return 0

    lax.fori_loop(0, nlocal, node_loop, 0)


def _edge_stage(q, kv, skip, srcpad, meta):
    mesh = plsc.VectorSubcoreMesh(core_axis_name="c", subcore_axis_name="s")
    f = pl.kernel(
        _edge_body,
        out_type=jax.ShapeDtypeStruct((N, HD), jnp.float32),
        mesh=mesh,
        scratch_types=[
            pltpu.VMEM((ROW,), jnp.float32),
            pltpu.VMEM((C, KVROW), jnp.float32),
            pltpu.VMEM((C,), jnp.int32),
            pltpu.VMEM((HD,), jnp.float32),
            pltpu.VMEM((ROW,), jnp.float32),
            pltpu.VMEM((H * L,), jnp.float32),
            pltpu.VMEM((2 * MP + L,), jnp.int32),
            pltpu.SemaphoreType.DMA,
        ],
    )
    return f(q, kv, skip, srcpad, meta)


def _prep_edges(edge_index):
    src = edge_index[0]
    dst = edge_index[1]
    order = jnp.argsort(dst)
    src_s = src[order]
    dst_s = dst[order]
    offs = jnp.searchsorted(dst_s, jnp.arange(N + 1, dtype=jnp.int32)).astype(jnp.int32)
    deg = offs[1:] - offs[:-1]
    cap = ((deg + (C - 1)) // C) * C
    offs_pad = jnp.concatenate(
        [jnp.zeros((1,), jnp.int32), jnp.cumsum(cap).astype(jnp.int32)]
    )
    pos = offs_pad[dst_s] + (jnp.arange(E, dtype=jnp.int32) - offs[dst_s])
    srcpad = jnp.zeros((LPAD,), jnp.int32).at[pos].set(src_s)

    def per_worker(a):
        a = jnp.pad(a, (0, NW * NPW - N)).reshape(NW, NPW)
        return jnp.pad(a, ((0, 0), (0, MP - NPW)))

    meta = jnp.concatenate(
        [per_worker(offs_pad[:N]), per_worker(deg),
         jnp.zeros((NW, L), jnp.int32)], axis=1
    )
    return srcpad, meta


def _layer(x, srcpad, meta, Wq, bq, Wk, bk, Wv, bv, Ws, bs):
    scale = 1.0 / (HD ** 0.5)
    q = _matmul(x, Wq * scale, bq * scale)
    kv = _matmul(x, jnp.concatenate([Wk, Wv], axis=1),
                 jnp.concatenate([bk, bv], axis=0))
    skip = _matmul(x, Ws, bs, bn=256)
    return _edge_stage(q, kv, skip, srcpad, meta)


def kernel(x, edge_index, Wq0, bq0, Wk0, bk0, Wv0, bv0, Ws0, bs0,
           Wq1, bq1, Wk1, bk1, Wv1, bv1, Ws1, bs1, Wout, bout):
    srcpad, meta = _prep_edges(edge_index)
    h = _layer(x, srcpad, meta, Wq0, bq0, Wk0, bk0, Wv0, bv0, Ws0, bs0)
    h = _layer(h, srcpad, meta, Wq1, bq1, Wk1, bk1, Wv1, bv1, Ws1, bs1)
    return _matmul(h, Wout, bout, bn=256)


# speedup vs baseline: 4.3191x; 1.6468x over previous
import jax
import jax.numpy as jnp
from jax import lax
from jax.experimental import pallas as pl
from jax.experimental.pallas import tpu as pltpu
from jax.experimental.pallas import tpu_sc as plsc

N = 10000
E = 160000
D = 256
HD = 256
H = 8
OUT = 256

NC = 2
NS = 16
NW = NC * NS
L = 16

NPW = (N + NW - 1) // NW
MP = ((NPW + 7) // 8) * 8
C = 8
LPAD = E + C * N
ROW = H * HD
KVROW = 2 * ROW


def _mm_body(x_ref, w_ref, b_ref, o_ref):
    o_ref[...] = (
        jnp.dot(x_ref[...], w_ref[...], preferred_element_type=jnp.float32)
        + b_ref[...]
    )


def _matmul(x, w, b, bm=1024, bn=1024):
    m, k = x.shape
    _, n = w.shape
    bn = min(bn, n)
    grid = (pl.cdiv(m, bm), pl.cdiv(n, bn))
    return pl.pallas_call(
        _mm_body,
        grid=grid,
        in_specs=[
            pl.BlockSpec((bm, k), lambda i, j: (i, 0)),
            pl.BlockSpec((k, bn), lambda i, j: (0, j)),
            pl.BlockSpec((1, bn), lambda i, j: (0, j)),
        ],
        out_specs=pl.BlockSpec((bm, bn), lambda i, j: (i, j)),
        out_shape=jax.ShapeDtypeStruct((m, n), jnp.float32),
    )(x, w, b.reshape(1, n))


def _lane_sum(v):
    i = lax.iota(jnp.int32, L)
    for s in (8, 4, 2, 1):
        idx = lax.bitwise_and(i + s, L - 1)
        v = v + v.at[idx].get(mode="promise_in_bounds")
    return v

def _edge_body(q_hbm, kv_hbm, skip_hbm, srcpad_hbm, meta_hbm, out_hbm,
               qrow_v, kvbuf_v, idx_v, outrow_v, acc_v, den_v, meta_v, sem):
    cid = lax.axis_index("c")
    sid = lax.axis_index("s")
    wid = sid * NC + cid
    n0 = wid * NPW
    nlocal = jnp.maximum(jnp.minimum(NPW, N - n0), 0)

    pltpu.sync_copy(meta_hbm.at[wid], meta_v)

    def node_loop(i, _):
        n = n0 + i
        base = meta_v[pl.ds(i, L)][0]
        deg = meta_v[pl.ds(MP + i, L)][0]

        pltpu.sync_copy(q_hbm.at[n], qrow_v)
        pltpu.sync_copy(skip_hbm.at[n], outrow_v)

        zero = jnp.zeros((L,), jnp.float32)
        for t in range(ROW // L):
            acc_v[pl.ds(L * t, L)] = zero
        for h in range(H):
            den_v[pl.ds(L * h, L)] = zero

        nchunks = (deg + (C - 1)) // C

        def chunk_loop(c, _):
            pos = pl.multiple_of(base + c * C, 8)
            pltpu.sync_copy(srcpad_hbm.at[pl.ds(pos, C)], idx_v)
            pltpu.sync_copy(kv_hbm.at[idx_v], kvbuf_v)
            cnt = jnp.minimum(deg - c * C, C)

            for h in range(H):
                qh = [qrow_v[pl.ds(h * HD + L * t, L)] for t in range(HD // L)]
                acch = [acc_v[pl.ds(h * HD + L * t, L)] for t in range(HD // L)]
                denh = den_v[pl.ds(L * h, L)]

                def edge_body(j, carry, h=h, qh=qh):
                    acch, denh = carry
                    dot = qh[0] * kvbuf_v[j, pl.ds(h * HD, L)]
                    for t in range(1, HD // L):
                        dot = dot + qh[t] * kvbuf_v[j, pl.ds(h * HD + L * t, L)]
                    p = jnp.exp(_lane_sum(dot))
                    denh = denh + p
                    acch = [
                        acch[t] + p * kvbuf_v[j, pl.ds(ROW + h * HD + L * t, L)]
                        for t in range(HD // L)
                    ]
                    return (acch, denh)

                acch, denh = lax.fori_loop(0, cnt, edge_body, (acch, denh))
                for t in range(HD // L):
                    acc_v[pl.ds(h * HD + L * t, L)] = acch[t]
                den_v[pl.ds(L * h, L)] = denh
            return 0

        lax.fori_loop(0, nchunks, chunk_loop, 0)

        for t in range(HD // L):
            o = outrow_v[pl.ds(L * t, L)]
            tot = jnp.zeros((L,), jnp.float32)
            for h in range(H):
                a = acc_v[pl.ds(h * HD + L * t, L)]
                d = den_v[pl.ds(L * h, L)]
                tot = tot + a / (d + 1e-16)
            o = o + tot * (1.0 / H)
            outrow_v[pl.ds(L * t, L)] = jnp.maximum(o, 0.0)
        pltpu.sync_copy(outrow_v, out_hbm.at[n])
        return 0

    lax.fori_loop(0, nlocal, node_loop, 0)


def _edge_stage(q, kv, skip, srcpad, meta):
    mesh = plsc.VectorSubcoreMesh(core_axis_name="c", subcore_axis_name="s")
    f = pl.kernel(
        _edge_body,
        out_type=jax.ShapeDtypeStruct((N, HD), jnp.float32),
        mesh=mesh,
        scratch_types=[
            pltpu.VMEM((ROW,), jnp.float32),
            pltpu.VMEM((C, KVROW), jnp.float32),
            pltpu.VMEM((C,), jnp.int32),
            pltpu.VMEM((HD,), jnp.float32),
            pltpu.VMEM((ROW,), jnp.float32),
            pltpu.VMEM((H * L,), jnp.float32),
            pltpu.VMEM((2 * MP + L,), jnp.int32),
            pltpu.SemaphoreType.DMA,
        ],
    )
    return f(q, kv, skip, srcpad, meta)


def _prep_edges(edge_index):
    src = edge_index[0]
    dst = edge_index[1]
    dst_s, src_s = lax.sort((dst, src), num_keys=1)
    ar = jnp.arange(E, dtype=jnp.int32)
    is_new = jnp.concatenate(
        [jnp.ones((1,), bool), dst_s[1:] != dst_s[:-1]]
    )
    seg_start = lax.cummax(jnp.where(is_new, ar, 0))
    rank = ar - seg_start
    prev_start = jnp.concatenate([jnp.zeros((1,), jnp.int32), seg_start[:-1]])
    prev_deg = jnp.where(is_new & (ar > 0), ar - prev_start, 0)
    pad_step = (-prev_deg) & (C - 1)
    pos = ar + jnp.cumsum(pad_step).astype(jnp.int32)
    srcpad = jnp.zeros((LPAD,), jnp.int32).at[pos].set(src_s)
    vals = jnp.stack([pos - rank, rank + 1], axis=1)
    nodeinfo = jnp.zeros((N, 2), jnp.int32).at[dst_s].max(vals)
    base_n = nodeinfo[:, 0]
    deg_n = nodeinfo[:, 1]

    def per_worker(a):
        a = jnp.pad(a, (0, NW * NPW - N)).reshape(NW, NPW)
        return jnp.pad(a, ((0, 0), (0, MP - NPW)))

    meta = jnp.concatenate(
        [per_worker(base_n), per_worker(deg_n),
         jnp.zeros((NW, L), jnp.int32)], axis=1
    )
    return srcpad, meta


def _layer(x, srcpad, meta, Wq, bq, Wk, bk, Wv, bv, Ws, bs):
    scale = 1.0 / (HD ** 0.5)
    q = _matmul(x, Wq * scale, bq * scale)
    kv = _matmul(x, jnp.concatenate([Wk, Wv], axis=1),
                 jnp.concatenate([bk, bv], axis=0))
    skip = _matmul(x, Ws, bs, bn=256)
    return _edge_stage(q, kv, skip, srcpad, meta)


def kernel(x, edge_index, Wq0, bq0, Wk0, bk0, Wv0, bv0, Ws0, bs0,
           Wq1, bq1, Wk1, bk1, Wv1, bv1, Ws1, bs1, Wout, bout):
    srcpad, meta = _prep_edges(edge_index)
    h = _layer(x, srcpad, meta, Wq0, bq0, Wk0, bk0, Wv0, bv0, Ws0, bs0)
    h = _layer(h, srcpad, meta, Wq1, bq1, Wk1, bk1, Wv1, bv1, Ws1, bs1)
    return _matmul(h, Wout, bout, bn=256)
